# baseline (speedup 1.0000x reference)
import functools

import jax
import jax.numpy as jnp
from jax import lax
from jax.experimental import pallas as pl
from jax.experimental.pallas import tpu as pltpu
from jax.experimental.pallas import tpu_sc as plsc

BATCH = 16384
DIM = 64
NC = 2
NS = 16
NW = NC * NS
BPW = BATCH // NW
CH = 128
NCH = BPW // CH
LANES = 16

_mesh = plsc.VectorSubcoreMesh(core_axis_name="c", subcore_axis_name="s")


@functools.partial(
    pl.kernel,
    mesh=_mesh,
    compiler_params=pltpu.CompilerParams(
        needs_layout_passes=False, use_tc_tiling_on_sc=False),
    out_type=jax.ShapeDtypeStruct((BATCH,), jnp.float32),
    scratch_types=[
        pltpu.VMEM((NCH, CH), jnp.int32),
        pltpu.VMEM((NCH, CH), jnp.int32),
        pltpu.VMEM((BPW, DIM), jnp.float32),
        pltpu.VMEM((BPW, DIM), jnp.float32),
        pltpu.VMEM((BPW,), jnp.float32),
        pltpu.VMEM((BPW,), jnp.float32),
        pltpu.VMEM((BPW,), jnp.float32),
        pltpu.SemaphoreType.DMA,
    ],
)
def _glove_sc(wi_hbm, ci_hbm, ww_hbm, wc_hbm, bw_hbm, bc_hbm, out_hbm,
              idx_w, idx_c, rows_w, rows_c, bias_w, bias_c, out_v, sem):
    wid = lax.axis_index("s") * NC + lax.axis_index("c")

    pltpu.sync_copy(wi_hbm.at[pl.ds(wid * NCH, NCH)], idx_w)
    pltpu.sync_copy(ci_hbm.at[pl.ds(wid * NCH, NCH)], idx_c)

    copies = []
    for j in range(NCH):
        sl = pl.ds(j * CH, CH)
        copies.append(pltpu.async_copy(ww_hbm.at[idx_w.at[j]], rows_w.at[sl], sem))
        copies.append(pltpu.async_copy(wc_hbm.at[idx_c.at[j]], rows_c.at[sl], sem))
        copies.append(pltpu.async_copy(bw_hbm.at[idx_w.at[j]], bias_w.at[sl], sem))
        copies.append(pltpu.async_copy(bc_hbm.at[idx_c.at[j]], bias_c.at[sl], sem))
    for c in copies:
        c.wait()

    lane = lax.iota(jnp.int32, LANES)

    def group(g, carry):
        rows = g * LANES + lane
        acc = bias_w[pl.ds(g * LANES, LANES)] + bias_c[pl.ds(g * LANES, LANES)]
        for d in range(DIM):
            col = jnp.full((LANES,), d, jnp.int32)
            acc = acc + (plsc.load_gather(rows_w, [rows, col])
                         * plsc.load_gather(rows_c, [rows, col]))
        out_v[pl.ds(g * LANES, LANES)] = acc
        return carry

    lax.fori_loop(0, BPW // LANES, group, 0)

    pltpu.sync_copy(out_v, out_hbm.at[pl.ds(wid * BPW, BPW)])


def kernel(word_idx, context_idx, W_word, W_ctx, b_word, b_ctx):
    wi = word_idx.astype(jnp.int32).reshape(BATCH // CH, CH)
    ci = context_idx.astype(jnp.int32).reshape(BATCH // CH, CH)
    out = _glove_sc(wi, ci, W_word, W_ctx,
                    b_word.reshape(-1), b_ctx.reshape(-1))
    return out.reshape(BATCH, 1)
